# baseline (speedup 1.0000x reference)
import jax
import jax.numpy as jnp
from jax import lax
from jax.experimental import pallas as pl
from jax.experimental.pallas import tpu as pltpu

B, N, D, C, IN, S = 64, 8192, 64, 256, 128, 3
CTRL_IN = IN + D
EPS = 1e-8

BB = 8
NB = 8192
GB = B // BB
GJ = N // NB

_DNK = (((1,), (0,)), ((), ()))


def _prologue_kernel(ctrl_in_ref, W_ctrl_ref, b_ctrl_ref,
                     Wk_r_ref, bk_r_ref, Wk_w_ref, bk_w_ref,
                     We_w_ref, be_w_ref, Wa_w_ref, ba_w_ref,
                     Wsc_r_ref, bsc_r_ref, Wsc_w_ref, bsc_w_ref,
                     h_ref, erase_ref, add_ref, Lm_ref, Lsq_ref,
                     par_r_ref, par_w_ref):
    f32 = jnp.float32
    h = jnp.maximum(
        jnp.dot(ctrl_in_ref[...], W_ctrl_ref[...],
                preferred_element_type=f32) + b_ctrl_ref[...], 0.0)
    h_ref[...] = h
    k_r = jnp.dot(h, Wk_r_ref[...], preferred_element_type=f32) + bk_r_ref[...]
    k_w = jnp.dot(h, Wk_w_ref[...], preferred_element_type=f32) + bk_w_ref[...]
    e = jax.nn.sigmoid(
        jnp.dot(h, We_w_ref[...], preferred_element_type=f32) + be_w_ref[...])
    a = jnp.tanh(
        jnp.dot(h, Wa_w_ref[...], preferred_element_type=f32) + ba_w_ref[...])
    erase_ref[...] = e
    add_ref[...] = a

    def pack_rows(rows):
        rs = [v[:, None, :] for v in rows]
        pad = jnp.zeros((B, 8 - len(rows), D), f32)
        return jnp.concatenate(rs + [pad], axis=1).astype(jnp.bfloat16)

    Lm_ref[...] = pack_rows([k_w, k_r, e * k_r, a, a * e])
    Lsq_ref[...] = pack_rows([jnp.ones((B, D), f32), e, e * e])

    ksq_r = jnp.sum(k_r * k_r, axis=-1, keepdims=True)
    ksq_w = jnp.sum(k_w * k_w, axis=-1, keepdims=True)
    ak = jnp.sum(a * k_r, axis=-1, keepdims=True)
    asq = jnp.sum(a * a, axis=-1, keepdims=True)
    for Wsc_ref, bsc_ref, ksq, extra, par_ref in (
            (Wsc_r_ref, bsc_r_ref, ksq_r, [ak, asq], par_r_ref),
            (Wsc_w_ref, bsc_w_ref, ksq_w, [], par_w_ref)):
        raw = (jnp.dot(h, Wsc_ref[...], preferred_element_type=f32)
               + bsc_ref[...])
        beta = jax.nn.softplus(raw[:, 0:1])
        g = jax.nn.sigmoid(raw[:, 1:2])
        gamma = jax.nn.softplus(raw[:, 2:3]) + 1.0
        slog = raw[:, 3:6]
        smax = jnp.max(slog, axis=-1, keepdims=True)
        sexp = jnp.exp(slog - smax)
        s = sexp / jnp.sum(sexp, axis=-1, keepdims=True)
        cols = [beta, g, gamma, s, ksq] + extra
        used = 7 + len(extra)
        cols.append(jnp.zeros((B, 128 - used), f32))
        par_ref[...] = jnp.concatenate(cols, axis=-1)


def _run_prologue(ctrl_in, W_ctrl, b_ctrl, Wk_r, bk_r, Wk_w, bk_w,
                  We_w, be_w, Wa_w, ba_w, Wsc_r, bsc_r, Wsc_w, bsc_w):
    out_shapes = (
        jax.ShapeDtypeStruct((B, C), jnp.float32),
        jax.ShapeDtypeStruct((B, D), jnp.float32),
        jax.ShapeDtypeStruct((B, D), jnp.float32),
        jax.ShapeDtypeStruct((B, 8, D), jnp.bfloat16),
        jax.ShapeDtypeStruct((B, 8, D), jnp.bfloat16),
        jax.ShapeDtypeStruct((B, 128), jnp.float32),
        jax.ShapeDtypeStruct((B, 128), jnp.float32),
    )
    return pl.pallas_call(
        _prologue_kernel,
        out_shape=out_shapes,
    )(ctrl_in, W_ctrl, b_ctrl, Wk_r, bk_r, Wk_w, bk_w,
      We_w, be_w, Wa_w, ba_w, Wsc_r, bsc_r, Wsc_w, bsc_w)


def _pass1_kernel(mem_ref, Lm_ref, Lsq_ref,
                  dkw_ref, dkr_ref, dekr_ref, da_ref, dae_ref,
                  ssq_ref, ssqe_ref, ssqee_ref):
    f32 = jnp.float32
    for b in range(BB):
        mb = mem_ref[b].astype(jnp.bfloat16)
        sq = mb * mb
        om = lax.dot_general(Lm_ref[b], mb, _DNK,
                             preferred_element_type=f32)
        osq = lax.dot_general(Lsq_ref[b], sq, _DNK,
                              preferred_element_type=f32)
        for r, ref in enumerate((dkw_ref, dkr_ref, dekr_ref, da_ref, dae_ref)):
            ref[b:b + 1, :] = om[r:r + 1, :]
        for r, ref in enumerate((ssq_ref, ssqe_ref, ssqee_ref)):
            ref[b:b + 1, :] = osq[r:r + 1, :]


def _run_pass1(mem_t, Lm, Lsq):
    big = pl.BlockSpec((BB, D, NB), lambda i, j: (i, 0, j))
    lspec = pl.BlockSpec((BB, 8, D), lambda i, j: (i, 0, 0))
    ospec = pl.BlockSpec((BB, NB), lambda i, j: (i, j))
    oshape = jax.ShapeDtypeStruct((B, N), jnp.float32)
    return pl.pallas_call(
        _pass1_kernel,
        grid=(GB, GJ),
        in_specs=[big, lspec, lspec],
        out_specs=[ospec] * 8,
        out_shape=[oshape] * 8,
        compiler_params=pltpu.CompilerParams(
            dimension_semantics=("parallel", "arbitrary")),
    )(mem_t, Lm, Lsq)


def _address(dots, sqn, par, pw):
    beta = par[:, 0:1]
    g = par[:, 1:2]
    gamma = par[:, 2:3]
    s0 = par[:, 3:4]
    s1 = par[:, 4:5]
    s2 = par[:, 5:6]
    knorm = jnp.sqrt(par[:, 6:7])
    norms = jnp.sqrt(jnp.maximum(sqn, 0.0)) * knorm
    z = beta * (dots / (norms + EPS))
    zmax = jnp.max(z, axis=-1, keepdims=True)
    ez = jnp.exp(z - zmax)
    wc = ez / jnp.sum(ez, axis=-1, keepdims=True)
    wg = g * wc + (1.0 - g) * pw
    roll_m1 = jnp.concatenate([wg[:, 1:], wg[:, :1]], axis=-1)
    roll_p1 = jnp.concatenate([wg[:, -1:], wg[:, :-1]], axis=-1)
    ws = s0 * roll_m1 + s1 * wg + s2 * roll_p1
    u = jnp.exp(gamma * jnp.log(ws))
    return u / (jnp.sum(u, axis=-1, keepdims=True) + EPS)


def _fin_w_kernel(dkw_ref, ssq_ref, par_ref, pw_ref, ww_ref):
    ww_ref[...] = _address(dkw_ref[...], ssq_ref[...], par_ref[...],
                           pw_ref[...])


def _run_fin_w(dkw, ssq, par_w, pw):
    row = pl.BlockSpec((BB, N), lambda i: (i, 0))
    return pl.pallas_call(
        _fin_w_kernel,
        grid=(GB,),
        in_specs=[row, row, pl.BlockSpec((BB, 128), lambda i: (i, 0)), row],
        out_specs=row,
        out_shape=jax.ShapeDtypeStruct((B, N), jnp.float32),
        compiler_params=pltpu.CompilerParams(
            dimension_semantics=("parallel",)),
    )(dkw, ssq, par_w, pw)


def _fin_r_kernel(dkr_ref, dekr_ref, da_ref, dae_ref,
                  ssq_ref, ssqe_ref, ssqee_ref,
                  ww_ref, par_ref, pr_ref,
                  wr_ref, wrww_ref, swr_ref):
    par = par_ref[...]
    ak = par[:, 7:8]
    asq = par[:, 8:9]
    ww = ww_ref[...]
    wwsq = ww * ww
    dots = dkr_ref[...] - ww * dekr_ref[...] + ww * ak
    sqn = (ssq_ref[...] - 2.0 * ww * ssqe_ref[...] + wwsq * ssqee_ref[...]
           + 2.0 * ww * da_ref[...] - 2.0 * wwsq * dae_ref[...] + wwsq * asq)
    wr = _address(dots, sqn, par, pr_ref[...])
    wr_ref[...] = wr
    p = wr * ww
    wrww_ref[...] = p
    swr_ref[...] = jnp.sum(p, axis=-1, keepdims=True)


def _run_fin_r(dkr, dekr, da, dae, ssq, ssqe, ssqee, ww, par_r, pr):
    row = pl.BlockSpec((BB, N), lambda i: (i, 0))
    return pl.pallas_call(
        _fin_r_kernel,
        grid=(GB,),
        in_specs=[row] * 7 + [row,
                              pl.BlockSpec((BB, 128), lambda i: (i, 0)), row],
        out_specs=[row, row, pl.BlockSpec((BB, 1), lambda i: (i, 0))],
        out_shape=[jax.ShapeDtypeStruct((B, N), jnp.float32),
                   jax.ShapeDtypeStruct((B, N), jnp.float32),
                   jax.ShapeDtypeStruct((B, 1), jnp.float32)],
        compiler_params=pltpu.CompilerParams(
            dimension_semantics=("parallel",)),
    )(dkr, dekr, da, dae, ssq, ssqe, ssqee, ww, par_r, pr)


def _pass2_kernel(mem_ref, wr_ref, wrww_ref, erase_ref, add_ref, swr_ref,
                  out_ref, acc1_ref, acc2_ref):
    j = pl.program_id(1)

    @pl.when(j == 0)
    def _init():
        acc1_ref[...] = jnp.zeros_like(acc1_ref)
        acc2_ref[...] = jnp.zeros_like(acc2_ref)

    for b in range(BB):
        mb = mem_ref[b]
        w1 = wr_ref[b:b + 1, :]
        w2 = wrww_ref[b:b + 1, :]
        p1 = mb * w1
        p2 = mb * w2
        f1 = sum([p1[:, k * 128:(k + 1) * 128] for k in range(NB // 128)])
        f2 = sum([p2[:, k * 128:(k + 1) * 128] for k in range(NB // 128)])
        acc1_ref[b] += f1
        acc2_ref[b] += f2

    @pl.when(j == GJ - 1)
    def _fin():
        cols1 = [jnp.sum(acc1_ref[b], axis=-1, keepdims=True)
                 for b in range(BB)]
        cols2 = [jnp.sum(acc2_ref[b], axis=-1, keepdims=True)
                 for b in range(BB)]
        p1t = jnp.concatenate(cols1, axis=-1).T
        p2t = jnp.concatenate(cols2, axis=-1).T
        out_ref[...] = (p1t - erase_ref[...] * p2t
                        + swr_ref[...] * add_ref[...])


def _run_pass2(mem_t, wr, wrww, erase, add, swr):
    big = pl.BlockSpec((BB, D, NB), lambda i, j: (i, 0, j))
    row = pl.BlockSpec((BB, NB), lambda i, j: (i, j))
    vec = pl.BlockSpec((BB, D), lambda i, j: (i, 0))
    return pl.pallas_call(
        _pass2_kernel,
        grid=(GB, GJ),
        in_specs=[big, row, row, vec, vec,
                  pl.BlockSpec((BB, 1), lambda i, j: (i, 0))],
        out_specs=vec,
        out_shape=jax.ShapeDtypeStruct((B, D), jnp.float32),
        scratch_shapes=[pltpu.VMEM((BB, D, 128), jnp.float32),
                        pltpu.VMEM((BB, D, 128), jnp.float32)],
        compiler_params=pltpu.CompilerParams(
            dimension_semantics=("parallel", "arbitrary")),
    )(mem_t, wr, wrww, erase, add, swr)


@jax.jit
def kernel(x, prev_memory, prev_read_weights, prev_write_weights,
           prev_read_vector,
           W_ctrl, b_ctrl,
           Wk_r, bk_r, Wb_r, bb_r, Wg_r, bg_r, Ws_r, bs_r, Wgam_r, bgam_r,
           Wk_w, bk_w, Wb_w, bb_w, Wg_w, bg_w, Ws_w, bs_w, Wgam_w, bgam_w,
           We_w, be_w, Wa_w, ba_w):
    ctrl_in = jnp.concatenate([x, prev_read_vector], axis=-1)

    def pack_scalar_weights(Wb, Wg, Wgam, Ws, bb, bg, bgam, bs):
        Wsc = jnp.concatenate([Wb, Wg, Wgam, Ws], axis=-1)
        Wsc = jnp.pad(Wsc, ((0, 0), (0, 122)))
        bsc = jnp.concatenate([bb, bg, bgam, bs], axis=-1)
        bsc = jnp.pad(bsc, (0, 122)).reshape(1, 128)
        return Wsc, bsc

    Wsc_r, bsc_r = pack_scalar_weights(Wb_r, Wg_r, Wgam_r, Ws_r,
                                       bb_r, bg_r, bgam_r, bs_r)
    Wsc_w, bsc_w = pack_scalar_weights(Wb_w, Wg_w, Wgam_w, Ws_w,
                                       bb_w, bg_w, bgam_w, bs_w)

    h, erase, add, Lm, Lsq, par_r, par_w = _run_prologue(
        ctrl_in, W_ctrl, b_ctrl.reshape(1, C),
        Wk_r, bk_r.reshape(1, D), Wk_w, bk_w.reshape(1, D),
        We_w, be_w.reshape(1, D), Wa_w, ba_w.reshape(1, D),
        Wsc_r, bsc_r, Wsc_w, bsc_w)

    mem_t = jnp.transpose(prev_memory, (0, 2, 1))

    dkw, dkr, dekr, da, dae, ssq, ssqe, ssqee = _run_pass1(mem_t, Lm, Lsq)
    ww = _run_fin_w(dkw, ssq, par_w, prev_write_weights)
    wr, wrww, swr = _run_fin_r(dkr, dekr, da, dae, ssq, ssqe, ssqee,
                               ww, par_r, prev_read_weights)
    read_vec = _run_pass2(mem_t, wr, wrww, erase, add, swr)
    return jnp.concatenate([h, read_vec], axis=-1)


# speedup vs baseline: 1.0094x; 1.0094x over previous
import jax
import jax.numpy as jnp
from jax import lax
from jax.experimental import pallas as pl
from jax.experimental.pallas import tpu as pltpu

B, N, D, C, IN, S = 64, 8192, 64, 256, 128, 3
CTRL_IN = IN + D
EPS = 1e-8

BB = 8
NB = 4096
GB = B // BB
GJ = N // NB

_DNK = (((1,), (0,)), ((), ()))


def _prologue_kernel(ctrl_in_ref, W_ctrl_ref, b_ctrl_ref,
                     Wk_r_ref, bk_r_ref, Wk_w_ref, bk_w_ref,
                     We_w_ref, be_w_ref, Wa_w_ref, ba_w_ref,
                     Wsc_r_ref, bsc_r_ref, Wsc_w_ref, bsc_w_ref,
                     h_ref, erase_ref, add_ref, Lm_ref, Lsq_ref,
                     par_r_ref, par_w_ref):
    f32 = jnp.float32
    h = jnp.maximum(
        jnp.dot(ctrl_in_ref[...], W_ctrl_ref[...],
                preferred_element_type=f32) + b_ctrl_ref[...], 0.0)
    h_ref[...] = h
    k_r = jnp.dot(h, Wk_r_ref[...], preferred_element_type=f32) + bk_r_ref[...]
    k_w = jnp.dot(h, Wk_w_ref[...], preferred_element_type=f32) + bk_w_ref[...]
    e = jax.nn.sigmoid(
        jnp.dot(h, We_w_ref[...], preferred_element_type=f32) + be_w_ref[...])
    a = jnp.tanh(
        jnp.dot(h, Wa_w_ref[...], preferred_element_type=f32) + ba_w_ref[...])
    erase_ref[...] = e
    add_ref[...] = a

    def pack_rows(rows):
        rs = [v[:, None, :] for v in rows]
        pad = jnp.zeros((B, 8 - len(rows), D), f32)
        return jnp.concatenate(rs + [pad], axis=1).astype(jnp.bfloat16)

    Lm_ref[...] = pack_rows([k_w, k_r, e * k_r, a, a * e])
    Lsq_ref[...] = pack_rows([jnp.ones((B, D), f32), e, e * e])

    ksq_r = jnp.sum(k_r * k_r, axis=-1, keepdims=True)
    ksq_w = jnp.sum(k_w * k_w, axis=-1, keepdims=True)
    ak = jnp.sum(a * k_r, axis=-1, keepdims=True)
    asq = jnp.sum(a * a, axis=-1, keepdims=True)
    for Wsc_ref, bsc_ref, ksq, extra, par_ref in (
            (Wsc_r_ref, bsc_r_ref, ksq_r, [ak, asq], par_r_ref),
            (Wsc_w_ref, bsc_w_ref, ksq_w, [], par_w_ref)):
        raw = (jnp.dot(h, Wsc_ref[...], preferred_element_type=f32)
               + bsc_ref[...])
        beta = jax.nn.softplus(raw[:, 0:1])
        g = jax.nn.sigmoid(raw[:, 1:2])
        gamma = jax.nn.softplus(raw[:, 2:3]) + 1.0
        slog = raw[:, 3:6]
        smax = jnp.max(slog, axis=-1, keepdims=True)
        sexp = jnp.exp(slog - smax)
        s = sexp / jnp.sum(sexp, axis=-1, keepdims=True)
        cols = [beta, g, gamma, s, ksq] + extra
        used = 7 + len(extra)
        cols.append(jnp.zeros((B, 128 - used), f32))
        par_ref[...] = jnp.concatenate(cols, axis=-1)


def _run_prologue(ctrl_in, W_ctrl, b_ctrl, Wk_r, bk_r, Wk_w, bk_w,
                  We_w, be_w, Wa_w, ba_w, Wsc_r, bsc_r, Wsc_w, bsc_w):
    out_shapes = (
        jax.ShapeDtypeStruct((B, C), jnp.float32),
        jax.ShapeDtypeStruct((B, D), jnp.float32),
        jax.ShapeDtypeStruct((B, D), jnp.float32),
        jax.ShapeDtypeStruct((B, 8, D), jnp.bfloat16),
        jax.ShapeDtypeStruct((B, 8, D), jnp.bfloat16),
        jax.ShapeDtypeStruct((B, 128), jnp.float32),
        jax.ShapeDtypeStruct((B, 128), jnp.float32),
    )
    return pl.pallas_call(
        _prologue_kernel,
        out_shape=out_shapes,
    )(ctrl_in, W_ctrl, b_ctrl, Wk_r, bk_r, Wk_w, bk_w,
      We_w, be_w, Wa_w, ba_w, Wsc_r, bsc_r, Wsc_w, bsc_w)


def _pass1_kernel(mem_ref, Lm_ref, Lsq_ref,
                  dkw_ref, dkr_ref, dekr_ref, da_ref, dae_ref,
                  ssq_ref, ssqe_ref, ssqee_ref):
    f32 = jnp.float32
    for b in range(BB):
        mb = mem_ref[b].astype(jnp.bfloat16)
        sq = mb * mb
        om = lax.dot_general(Lm_ref[b], mb, _DNK,
                             preferred_element_type=f32)
        osq = lax.dot_general(Lsq_ref[b], sq, _DNK,
                              preferred_element_type=f32)
        for r, ref in enumerate((dkw_ref, dkr_ref, dekr_ref, da_ref, dae_ref)):
            ref[b:b + 1, :] = om[r:r + 1, :]
        for r, ref in enumerate((ssq_ref, ssqe_ref, ssqee_ref)):
            ref[b:b + 1, :] = osq[r:r + 1, :]


def _run_pass1(mem_t, Lm, Lsq):
    big = pl.BlockSpec((BB, D, NB), lambda i, j: (i, 0, j))
    lspec = pl.BlockSpec((BB, 8, D), lambda i, j: (i, 0, 0))
    ospec = pl.BlockSpec((BB, NB), lambda i, j: (i, j))
    oshape = jax.ShapeDtypeStruct((B, N), jnp.float32)
    return pl.pallas_call(
        _pass1_kernel,
        grid=(GB, GJ),
        in_specs=[big, lspec, lspec],
        out_specs=[ospec] * 8,
        out_shape=[oshape] * 8,
        compiler_params=pltpu.CompilerParams(
            dimension_semantics=("parallel", "arbitrary")),
    )(mem_t, Lm, Lsq)


def _address(dots, sqn, par, pw):
    beta = par[:, 0:1]
    g = par[:, 1:2]
    gamma = par[:, 2:3]
    s0 = par[:, 3:4]
    s1 = par[:, 4:5]
    s2 = par[:, 5:6]
    knorm = jnp.sqrt(par[:, 6:7])
    norms = jnp.sqrt(jnp.maximum(sqn, 0.0)) * knorm
    z = beta * (dots / (norms + EPS))
    zmax = jnp.max(z, axis=-1, keepdims=True)
    ez = jnp.exp(z - zmax)
    wc = ez / jnp.sum(ez, axis=-1, keepdims=True)
    wg = g * wc + (1.0 - g) * pw
    roll_m1 = jnp.concatenate([wg[:, 1:], wg[:, :1]], axis=-1)
    roll_p1 = jnp.concatenate([wg[:, -1:], wg[:, :-1]], axis=-1)
    ws = s0 * roll_m1 + s1 * wg + s2 * roll_p1
    u = jnp.exp(gamma * jnp.log(ws))
    return u / (jnp.sum(u, axis=-1, keepdims=True) + EPS)


def _fin_w_kernel(dkw_ref, ssq_ref, par_ref, pw_ref, ww_ref):
    ww_ref[...] = _address(dkw_ref[...], ssq_ref[...], par_ref[...],
                           pw_ref[...])


def _run_fin_w(dkw, ssq, par_w, pw):
    row = pl.BlockSpec((BB, N), lambda i: (i, 0))
    return pl.pallas_call(
        _fin_w_kernel,
        grid=(GB,),
        in_specs=[row, row, pl.BlockSpec((BB, 128), lambda i: (i, 0)), row],
        out_specs=row,
        out_shape=jax.ShapeDtypeStruct((B, N), jnp.float32),
        compiler_params=pltpu.CompilerParams(
            dimension_semantics=("parallel",)),
    )(dkw, ssq, par_w, pw)


def _fin_r_kernel(dkr_ref, dekr_ref, da_ref, dae_ref,
                  ssq_ref, ssqe_ref, ssqee_ref,
                  ww_ref, par_ref, pr_ref,
                  wr_ref, wrww_ref, swr_ref):
    par = par_ref[...]
    ak = par[:, 7:8]
    asq = par[:, 8:9]
    ww = ww_ref[...]
    wwsq = ww * ww
    dots = dkr_ref[...] - ww * dekr_ref[...] + ww * ak
    sqn = (ssq_ref[...] - 2.0 * ww * ssqe_ref[...] + wwsq * ssqee_ref[...]
           + 2.0 * ww * da_ref[...] - 2.0 * wwsq * dae_ref[...] + wwsq * asq)
    wr = _address(dots, sqn, par, pr_ref[...])
    wr_ref[...] = wr
    p = wr * ww
    wrww_ref[...] = p
    swr_ref[...] = jnp.sum(p, axis=-1, keepdims=True)


def _run_fin_r(dkr, dekr, da, dae, ssq, ssqe, ssqee, ww, par_r, pr):
    row = pl.BlockSpec((BB, N), lambda i: (i, 0))
    return pl.pallas_call(
        _fin_r_kernel,
        grid=(GB,),
        in_specs=[row] * 7 + [row,
                              pl.BlockSpec((BB, 128), lambda i: (i, 0)), row],
        out_specs=[row, row, pl.BlockSpec((BB, 1), lambda i: (i, 0))],
        out_shape=[jax.ShapeDtypeStruct((B, N), jnp.float32),
                   jax.ShapeDtypeStruct((B, N), jnp.float32),
                   jax.ShapeDtypeStruct((B, 1), jnp.float32)],
        compiler_params=pltpu.CompilerParams(
            dimension_semantics=("parallel",)),
    )(dkr, dekr, da, dae, ssq, ssqe, ssqee, ww, par_r, pr)


def _pass2_kernel(mem_ref, wr_ref, wrww_ref, erase_ref, add_ref, swr_ref,
                  out_ref, acc1_ref, acc2_ref):
    j = pl.program_id(1)

    @pl.when(j == 0)
    def _init():
        acc1_ref[...] = jnp.zeros_like(acc1_ref)
        acc2_ref[...] = jnp.zeros_like(acc2_ref)

    for b in range(BB):
        mb = mem_ref[b]
        w1 = wr_ref[b:b + 1, :]
        w2 = wrww_ref[b:b + 1, :]
        p1 = mb * w1
        p2 = mb * w2
        f1 = sum([p1[:, k * 128:(k + 1) * 128] for k in range(NB // 128)])
        f2 = sum([p2[:, k * 128:(k + 1) * 128] for k in range(NB // 128)])
        acc1_ref[b] += f1
        acc2_ref[b] += f2

    @pl.when(j == GJ - 1)
    def _fin():
        cols1 = [jnp.sum(acc1_ref[b], axis=-1, keepdims=True)
                 for b in range(BB)]
        cols2 = [jnp.sum(acc2_ref[b], axis=-1, keepdims=True)
                 for b in range(BB)]
        p1t = jnp.concatenate(cols1, axis=-1).T
        p2t = jnp.concatenate(cols2, axis=-1).T
        out_ref[...] = (p1t - erase_ref[...] * p2t
                        + swr_ref[...] * add_ref[...])


def _run_pass2(mem_t, wr, wrww, erase, add, swr):
    big = pl.BlockSpec((BB, D, NB), lambda i, j: (i, 0, j))
    row = pl.BlockSpec((BB, NB), lambda i, j: (i, j))
    vec = pl.BlockSpec((BB, D), lambda i, j: (i, 0))
    return pl.pallas_call(
        _pass2_kernel,
        grid=(GB, GJ),
        in_specs=[big, row, row, vec, vec,
                  pl.BlockSpec((BB, 1), lambda i, j: (i, 0))],
        out_specs=vec,
        out_shape=jax.ShapeDtypeStruct((B, D), jnp.float32),
        scratch_shapes=[pltpu.VMEM((BB, D, 128), jnp.float32),
                        pltpu.VMEM((BB, D, 128), jnp.float32)],
        compiler_params=pltpu.CompilerParams(
            dimension_semantics=("parallel", "arbitrary")),
    )(mem_t, wr, wrww, erase, add, swr)


@jax.jit
def kernel(x, prev_memory, prev_read_weights, prev_write_weights,
           prev_read_vector,
           W_ctrl, b_ctrl,
           Wk_r, bk_r, Wb_r, bb_r, Wg_r, bg_r, Ws_r, bs_r, Wgam_r, bgam_r,
           Wk_w, bk_w, Wb_w, bb_w, Wg_w, bg_w, Ws_w, bs_w, Wgam_w, bgam_w,
           We_w, be_w, Wa_w, ba_w):
    ctrl_in = jnp.concatenate([x, prev_read_vector], axis=-1)

    def pack_scalar_weights(Wb, Wg, Wgam, Ws, bb, bg, bgam, bs):
        Wsc = jnp.concatenate([Wb, Wg, Wgam, Ws], axis=-1)
        Wsc = jnp.pad(Wsc, ((0, 0), (0, 122)))
        bsc = jnp.concatenate([bb, bg, bgam, bs], axis=-1)
        bsc = jnp.pad(bsc, (0, 122)).reshape(1, 128)
        return Wsc, bsc

    Wsc_r, bsc_r = pack_scalar_weights(Wb_r, Wg_r, Wgam_r, Ws_r,
                                       bb_r, bg_r, bgam_r, bs_r)
    Wsc_w, bsc_w = pack_scalar_weights(Wb_w, Wg_w, Wgam_w, Ws_w,
                                       bb_w, bg_w, bgam_w, bs_w)

    h, erase, add, Lm, Lsq, par_r, par_w = _run_prologue(
        ctrl_in, W_ctrl, b_ctrl.reshape(1, C),
        Wk_r, bk_r.reshape(1, D), Wk_w, bk_w.reshape(1, D),
        We_w, be_w.reshape(1, D), Wa_w, ba_w.reshape(1, D),
        Wsc_r, bsc_r, Wsc_w, bsc_w)

    mem_t = jnp.transpose(prev_memory, (0, 2, 1))

    dkw, dkr, dekr, da, dae, ssq, ssqe, ssqee = _run_pass1(mem_t, Lm, Lsq)
    ww = _run_fin_w(dkw, ssq, par_w, prev_write_weights)
    wr, wrww, swr = _run_fin_r(dkr, dekr, da, dae, ssq, ssqe, ssqee,
                               ww, par_r, prev_read_weights)
    read_vec = _run_pass2(mem_t, wr, wrww, erase, add, swr)
    return jnp.concatenate([h, read_vec], axis=-1)


# speedup vs baseline: 1.0663x; 1.0564x over previous
import jax
import jax.numpy as jnp
from jax import lax
from jax.experimental import pallas as pl
from jax.experimental.pallas import tpu as pltpu

B, N, D, C, IN, S = 64, 8192, 64, 256, 128, 3
CTRL_IN = IN + D
EPS = 1e-8

BB = 8
NB = 4096
GB = B // BB
GJ = N // NB

_DNK = (((1,), (0,)), ((), ()))


def _prologue_kernel(ctrl_in_ref, W_ctrl_ref, b_ctrl_ref,
                     Wk_r_ref, bk_r_ref, Wk_w_ref, bk_w_ref,
                     We_w_ref, be_w_ref, Wa_w_ref, ba_w_ref,
                     Wsc_r_ref, bsc_r_ref, Wsc_w_ref, bsc_w_ref,
                     h_ref, erase_ref, add_ref, Lm_ref, Lsq_ref,
                     par_r_ref, par_w_ref):
    f32 = jnp.float32
    h = jnp.maximum(
        jnp.dot(ctrl_in_ref[...], W_ctrl_ref[...],
                preferred_element_type=f32) + b_ctrl_ref[...], 0.0)
    h_ref[...] = h
    k_r = jnp.dot(h, Wk_r_ref[...], preferred_element_type=f32) + bk_r_ref[...]
    k_w = jnp.dot(h, Wk_w_ref[...], preferred_element_type=f32) + bk_w_ref[...]
    e = jax.nn.sigmoid(
        jnp.dot(h, We_w_ref[...], preferred_element_type=f32) + be_w_ref[...])
    a = jnp.tanh(
        jnp.dot(h, Wa_w_ref[...], preferred_element_type=f32) + ba_w_ref[...])
    erase_ref[...] = e
    add_ref[...] = a

    def pack_rows(rows):
        rs = [v[:, None, :] for v in rows]
        pad = jnp.zeros((B, 8 - len(rows), D), f32)
        return jnp.concatenate(rs + [pad], axis=1).astype(jnp.bfloat16)

    Lm_ref[...] = pack_rows([k_w, k_r, e * k_r, a, a * e])
    Lsq_ref[...] = pack_rows([jnp.ones((B, D), f32), e, e * e])

    ksq_r = jnp.sum(k_r * k_r, axis=-1, keepdims=True)
    ksq_w = jnp.sum(k_w * k_w, axis=-1, keepdims=True)
    ak = jnp.sum(a * k_r, axis=-1, keepdims=True)
    asq = jnp.sum(a * a, axis=-1, keepdims=True)
    for Wsc_ref, bsc_ref, ksq, extra, par_ref in (
            (Wsc_r_ref, bsc_r_ref, ksq_r, [ak, asq], par_r_ref),
            (Wsc_w_ref, bsc_w_ref, ksq_w, [], par_w_ref)):
        raw = (jnp.dot(h, Wsc_ref[...], preferred_element_type=f32)
               + bsc_ref[...])
        beta = jax.nn.softplus(raw[:, 0:1])
        g = jax.nn.sigmoid(raw[:, 1:2])
        gamma = jax.nn.softplus(raw[:, 2:3]) + 1.0
        slog = raw[:, 3:6]
        smax = jnp.max(slog, axis=-1, keepdims=True)
        sexp = jnp.exp(slog - smax)
        s = sexp / jnp.sum(sexp, axis=-1, keepdims=True)
        cols = [beta, g, gamma, s, ksq] + extra
        used = 7 + len(extra)
        cols.append(jnp.zeros((B, 128 - used), f32))
        par_ref[...] = jnp.concatenate(cols, axis=-1)


def _run_prologue(ctrl_in, W_ctrl, b_ctrl, Wk_r, bk_r, Wk_w, bk_w,
                  We_w, be_w, Wa_w, ba_w, Wsc_r, bsc_r, Wsc_w, bsc_w):
    out_shapes = (
        jax.ShapeDtypeStruct((B, C), jnp.float32),
        jax.ShapeDtypeStruct((B, D), jnp.float32),
        jax.ShapeDtypeStruct((B, D), jnp.float32),
        jax.ShapeDtypeStruct((B, 8, D), jnp.bfloat16),
        jax.ShapeDtypeStruct((B, 8, D), jnp.bfloat16),
        jax.ShapeDtypeStruct((B, 128), jnp.float32),
        jax.ShapeDtypeStruct((B, 128), jnp.float32),
    )
    return pl.pallas_call(
        _prologue_kernel,
        out_shape=out_shapes,
    )(ctrl_in, W_ctrl, b_ctrl, Wk_r, bk_r, Wk_w, bk_w,
      We_w, be_w, Wa_w, ba_w, Wsc_r, bsc_r, Wsc_w, bsc_w)


def _pass1_kernel(mem_ref, Lm_ref, Lsq_ref, par_w_ref, pw_ref,
                  dkr_ref, dekr_ref, da_ref, dae_ref,
                  ssq_ref, ssqe_ref, ssqee_ref, ww_ref,
                  dkw_s, ssq_s):
    f32 = jnp.float32
    j = pl.program_id(1)
    off = pl.multiple_of(j * NB, NB)
    for b in range(BB):
        mb = mem_ref[b].astype(jnp.bfloat16)
        sq = mb * mb
        om = lax.dot_general(Lm_ref[b], mb, _DNK,
                             preferred_element_type=f32)
        osq = lax.dot_general(Lsq_ref[b], sq, _DNK,
                              preferred_element_type=f32)
        dkw_s[b:b + 1, pl.ds(off, NB)] = om[0:1, :]
        for r, ref in enumerate((dkr_ref, dekr_ref, da_ref, dae_ref), 1):
            ref[b:b + 1, :] = om[r:r + 1, :]
        ssq_s[b:b + 1, pl.ds(off, NB)] = osq[0:1, :]
        for r, ref in enumerate((ssqe_ref, ssqee_ref), 1):
            ref[b:b + 1, :] = osq[r:r + 1, :]
        ssq_ref[b:b + 1, :] = osq[0:1, :]

    @pl.when(j == GJ - 1)
    def _fin():
        ww_ref[...] = _address(dkw_s[...], ssq_s[...], par_w_ref[...],
                               pw_ref[...])


def _run_pass1(mem_t, Lm, Lsq, par_w, pw):
    big = pl.BlockSpec((BB, D, NB), lambda i, j: (i, 0, j))
    lspec = pl.BlockSpec((BB, 8, D), lambda i, j: (i, 0, 0))
    ospec = pl.BlockSpec((BB, NB), lambda i, j: (i, j))
    oshape = jax.ShapeDtypeStruct((B, N), jnp.float32)
    rowspec = pl.BlockSpec((BB, N), lambda i, j: (i, 0))
    return pl.pallas_call(
        _pass1_kernel,
        grid=(GB, GJ),
        in_specs=[big, lspec, lspec,
                  pl.BlockSpec((BB, 128), lambda i, j: (i, 0)), rowspec],
        out_specs=[ospec] * 7 + [rowspec],
        out_shape=[oshape] * 7 + [jax.ShapeDtypeStruct((B, N), jnp.float32)],
        scratch_shapes=[pltpu.VMEM((BB, N), jnp.float32),
                        pltpu.VMEM((BB, N), jnp.float32)],
        compiler_params=pltpu.CompilerParams(
            dimension_semantics=("parallel", "arbitrary")),
    )(mem_t, Lm, Lsq, par_w, pw)


def _address(dots, sqn, par, pw):
    beta = par[:, 0:1]
    g = par[:, 1:2]
    gamma = par[:, 2:3]
    s0 = par[:, 3:4]
    s1 = par[:, 4:5]
    s2 = par[:, 5:6]
    knorm = jnp.sqrt(par[:, 6:7])
    norms = jnp.sqrt(jnp.maximum(sqn, 0.0)) * knorm
    z = beta * (dots / (norms + EPS))
    zmax = jnp.max(z, axis=-1, keepdims=True)
    ez = jnp.exp(z - zmax)
    wc = ez / jnp.sum(ez, axis=-1, keepdims=True)
    wg = g * wc + (1.0 - g) * pw
    roll_m1 = jnp.concatenate([wg[:, 1:], wg[:, :1]], axis=-1)
    roll_p1 = jnp.concatenate([wg[:, -1:], wg[:, :-1]], axis=-1)
    ws = s0 * roll_m1 + s1 * wg + s2 * roll_p1
    u = jnp.exp(gamma * jnp.log(ws))
    return u / (jnp.sum(u, axis=-1, keepdims=True) + EPS)


def _pass2_kernel(mem_ref, dkr_ref, dekr_ref, da_ref, dae_ref,
                  ssq_ref, ssqe_ref, ssqee_ref, ww_ref, par_ref, pr_ref,
                  erase_ref, add_ref,
                  out_ref, acc1_ref, acc2_ref, wr_s, wrww_s, swr_s):
    j = pl.program_id(1)

    @pl.when(j == 0)
    def _finalize_read_head():
        par = par_ref[...]
        ak = par[:, 7:8]
        asq = par[:, 8:9]
        ww = ww_ref[...]
        wwsq = ww * ww
        dots = dkr_ref[...] - ww * dekr_ref[...] + ww * ak
        sqn = (ssq_ref[...] - 2.0 * ww * ssqe_ref[...] + wwsq * ssqee_ref[...]
               + 2.0 * ww * da_ref[...] - 2.0 * wwsq * dae_ref[...]
               + wwsq * asq)
        wr = _address(dots, sqn, par, pr_ref[...])
        wr_s[...] = wr
        p = wr * ww
        wrww_s[...] = p
        swr_s[...] = jnp.sum(p, axis=-1, keepdims=True)
        acc1_ref[...] = jnp.zeros_like(acc1_ref)
        acc2_ref[...] = jnp.zeros_like(acc2_ref)

    off = pl.multiple_of(j * NB, NB)
    for b in range(BB):
        mb = mem_ref[b]
        w1 = wr_s[b:b + 1, pl.ds(off, NB)]
        w2 = wrww_s[b:b + 1, pl.ds(off, NB)]
        p1 = mb * w1
        p2 = mb * w2
        f1 = sum([p1[:, k * 128:(k + 1) * 128] for k in range(NB // 128)])
        f2 = sum([p2[:, k * 128:(k + 1) * 128] for k in range(NB // 128)])
        acc1_ref[b] += f1
        acc2_ref[b] += f2

    @pl.when(j == GJ - 1)
    def _fin():
        cols1 = [jnp.sum(acc1_ref[b], axis=-1, keepdims=True)
                 for b in range(BB)]
        cols2 = [jnp.sum(acc2_ref[b], axis=-1, keepdims=True)
                 for b in range(BB)]
        p1t = jnp.concatenate(cols1, axis=-1).T
        p2t = jnp.concatenate(cols2, axis=-1).T
        out_ref[...] = (p1t - erase_ref[...] * p2t
                        + swr_s[...] * add_ref[...])


def _run_pass2(mem_t, dkr, dekr, da, dae, ssq, ssqe, ssqee, ww, par_r, pr,
               erase, add):
    big = pl.BlockSpec((BB, D, NB), lambda i, j: (i, 0, j))
    rowspec = pl.BlockSpec((BB, N), lambda i, j: (i, 0))
    vec = pl.BlockSpec((BB, D), lambda i, j: (i, 0))
    return pl.pallas_call(
        _pass2_kernel,
        grid=(GB, GJ),
        in_specs=[big] + [rowspec] * 8
        + [pl.BlockSpec((BB, 128), lambda i, j: (i, 0)), rowspec, vec, vec],
        out_specs=vec,
        out_shape=jax.ShapeDtypeStruct((B, D), jnp.float32),
        scratch_shapes=[pltpu.VMEM((BB, D, 128), jnp.float32),
                        pltpu.VMEM((BB, D, 128), jnp.float32),
                        pltpu.VMEM((BB, N), jnp.float32),
                        pltpu.VMEM((BB, N), jnp.float32),
                        pltpu.VMEM((BB, 1), jnp.float32)],
        compiler_params=pltpu.CompilerParams(
            dimension_semantics=("parallel", "arbitrary")),
    )(mem_t, dkr, dekr, da, dae, ssq, ssqe, ssqee, ww, par_r, pr, erase, add)


@jax.jit
def kernel(x, prev_memory, prev_read_weights, prev_write_weights,
           prev_read_vector,
           W_ctrl, b_ctrl,
           Wk_r, bk_r, Wb_r, bb_r, Wg_r, bg_r, Ws_r, bs_r, Wgam_r, bgam_r,
           Wk_w, bk_w, Wb_w, bb_w, Wg_w, bg_w, Ws_w, bs_w, Wgam_w, bgam_w,
           We_w, be_w, Wa_w, ba_w):
    ctrl_in = jnp.concatenate([x, prev_read_vector], axis=-1)

    def pack_scalar_weights(Wb, Wg, Wgam, Ws, bb, bg, bgam, bs):
        Wsc = jnp.concatenate([Wb, Wg, Wgam, Ws], axis=-1)
        Wsc = jnp.pad(Wsc, ((0, 0), (0, 122)))
        bsc = jnp.concatenate([bb, bg, bgam, bs], axis=-1)
        bsc = jnp.pad(bsc, (0, 122)).reshape(1, 128)
        return Wsc, bsc

    Wsc_r, bsc_r = pack_scalar_weights(Wb_r, Wg_r, Wgam_r, Ws_r,
                                       bb_r, bg_r, bgam_r, bs_r)
    Wsc_w, bsc_w = pack_scalar_weights(Wb_w, Wg_w, Wgam_w, Ws_w,
                                       bb_w, bg_w, bgam_w, bs_w)

    h, erase, add, Lm, Lsq, par_r, par_w = _run_prologue(
        ctrl_in, W_ctrl, b_ctrl.reshape(1, C),
        Wk_r, bk_r.reshape(1, D), Wk_w, bk_w.reshape(1, D),
        We_w, be_w.reshape(1, D), Wa_w, ba_w.reshape(1, D),
        Wsc_r, bsc_r, Wsc_w, bsc_w)

    mem_t = jnp.transpose(prev_memory, (0, 2, 1))

    dkr, dekr, da, dae, ssq, ssqe, ssqee, ww = _run_pass1(
        mem_t, Lm, Lsq, par_w, prev_write_weights)
    read_vec = _run_pass2(mem_t, dkr, dekr, da, dae, ssq, ssqe, ssqee,
                          ww, par_r, prev_read_weights, erase, add)
    return jnp.concatenate([h, read_vec], axis=-1)


# speedup vs baseline: 1.1203x; 1.0507x over previous
import jax
import jax.numpy as jnp
from jax import lax
from jax.experimental import pallas as pl
from jax.experimental.pallas import tpu as pltpu

B, N, D, C, IN, S = 64, 8192, 64, 256, 128, 3
CTRL_IN = IN + D
EPS = 1e-8

BB = 8
NB = 4096
GB = B // BB
GJ = N // NB

_DNK = (((1,), (0,)), ((), ()))


def _prologue_kernel(ctrl_in_ref, W_ctrl_ref, b_ctrl_ref,
                     Wk_r_ref, bk_r_ref, Wk_w_ref, bk_w_ref,
                     We_w_ref, be_w_ref, Wa_w_ref, ba_w_ref,
                     Wsc_r_ref, bsc_r_ref, Wsc_w_ref, bsc_w_ref,
                     h_ref, erase_ref, add_ref, Lm_ref, Lsq_ref,
                     par_r_ref, par_w_ref):
    f32 = jnp.float32
    h = jnp.maximum(
        jnp.dot(ctrl_in_ref[...], W_ctrl_ref[...],
                preferred_element_type=f32) + b_ctrl_ref[...], 0.0)
    h_ref[...] = h
    k_r = jnp.dot(h, Wk_r_ref[...], preferred_element_type=f32) + bk_r_ref[...]
    k_w = jnp.dot(h, Wk_w_ref[...], preferred_element_type=f32) + bk_w_ref[...]
    e = jax.nn.sigmoid(
        jnp.dot(h, We_w_ref[...], preferred_element_type=f32) + be_w_ref[...])
    a = jnp.tanh(
        jnp.dot(h, Wa_w_ref[...], preferred_element_type=f32) + ba_w_ref[...])
    erase_ref[...] = e
    add_ref[...] = a

    def pack_rows(rows):
        rs = [v[:, None, :] for v in rows]
        pad = jnp.zeros((B, 8 - len(rows), D), f32)
        return jnp.concatenate(rs + [pad], axis=1).astype(jnp.bfloat16)

    Lm_ref[...] = pack_rows([k_w, k_r, e * k_r, a, a * e])
    Lsq_ref[...] = pack_rows([jnp.ones((B, D), f32), e, e * e])

    ksq_r = jnp.sum(k_r * k_r, axis=-1, keepdims=True)
    ksq_w = jnp.sum(k_w * k_w, axis=-1, keepdims=True)
    ak = jnp.sum(a * k_r, axis=-1, keepdims=True)
    asq = jnp.sum(a * a, axis=-1, keepdims=True)
    for Wsc_ref, bsc_ref, ksq, extra, par_ref in (
            (Wsc_r_ref, bsc_r_ref, ksq_r, [ak, asq], par_r_ref),
            (Wsc_w_ref, bsc_w_ref, ksq_w, [], par_w_ref)):
        raw = (jnp.dot(h, Wsc_ref[...], preferred_element_type=f32)
               + bsc_ref[...])
        beta = jax.nn.softplus(raw[:, 0:1])
        g = jax.nn.sigmoid(raw[:, 1:2])
        gamma = jax.nn.softplus(raw[:, 2:3]) + 1.0
        slog = raw[:, 3:6]
        smax = jnp.max(slog, axis=-1, keepdims=True)
        sexp = jnp.exp(slog - smax)
        s = sexp / jnp.sum(sexp, axis=-1, keepdims=True)
        cols = [beta, g, gamma, s, ksq] + extra
        used = 7 + len(extra)
        cols.append(jnp.zeros((B, 128 - used), f32))
        par_ref[...] = jnp.concatenate(cols, axis=-1)


def _run_prologue(ctrl_in, W_ctrl, b_ctrl, Wk_r, bk_r, Wk_w, bk_w,
                  We_w, be_w, Wa_w, ba_w, Wsc_r, bsc_r, Wsc_w, bsc_w):
    out_shapes = (
        jax.ShapeDtypeStruct((B, C), jnp.float32),
        jax.ShapeDtypeStruct((B, D), jnp.float32),
        jax.ShapeDtypeStruct((B, D), jnp.float32),
        jax.ShapeDtypeStruct((B, 8, D), jnp.bfloat16),
        jax.ShapeDtypeStruct((B, 8, D), jnp.bfloat16),
        jax.ShapeDtypeStruct((B, 128), jnp.float32),
        jax.ShapeDtypeStruct((B, 128), jnp.float32),
    )
    return pl.pallas_call(
        _prologue_kernel,
        out_shape=out_shapes,
    )(ctrl_in, W_ctrl, b_ctrl, Wk_r, bk_r, Wk_w, bk_w,
      We_w, be_w, Wa_w, ba_w, Wsc_r, bsc_r, Wsc_w, bsc_w)


def _pass1_kernel(mem_ref, Lm_ref, Lsq_ref, par_w_ref, pw_ref,
                  par_r_ref, pr_ref,
                  wr_ref, wrww_ref, swr_ref, fam_s):
    f32 = jnp.float32
    j = pl.program_id(1)
    off = pl.multiple_of(j * NB, NB)
    for b in range(BB):
        mb = mem_ref[b].astype(jnp.bfloat16)
        sq = mb * mb
        om = lax.dot_general(Lm_ref[b], mb, _DNK,
                             preferred_element_type=f32)
        osq = lax.dot_general(Lsq_ref[b], sq, _DNK,
                              preferred_element_type=f32)
        fam_s[0, b:b + 1, pl.ds(off, NB)] = om[0:1, :]
        fam_s[1, b:b + 1, pl.ds(off, NB)] = om[1:2, :]
        fam_s[2, b:b + 1, pl.ds(off, NB)] = om[2:3, :]
        fam_s[3, b:b + 1, pl.ds(off, NB)] = om[3:4, :]
        fam_s[4, b:b + 1, pl.ds(off, NB)] = om[4:5, :]
        fam_s[5, b:b + 1, pl.ds(off, NB)] = osq[0:1, :]
        fam_s[6, b:b + 1, pl.ds(off, NB)] = osq[1:2, :]
        fam_s[7, b:b + 1, pl.ds(off, NB)] = osq[2:3, :]

    @pl.when(j == GJ - 1)
    def _fin():
        par_r = par_r_ref[...]
        ak = par_r[:, 7:8]
        asq = par_r[:, 8:9]
        ww = _address(fam_s[0], fam_s[5], par_w_ref[...], pw_ref[...])
        wwsq = ww * ww
        dots = fam_s[1] - ww * fam_s[2] + ww * ak
        sqn = (fam_s[5] - 2.0 * ww * fam_s[6] + wwsq * fam_s[7]
               + 2.0 * ww * fam_s[3] - 2.0 * wwsq * fam_s[4] + wwsq * asq)
        wr = _address(dots, sqn, par_r, pr_ref[...])
        wr_ref[...] = wr
        p = wr * ww
        wrww_ref[...] = p
        swr_ref[...] = jnp.sum(p, axis=-1, keepdims=True)


def _run_pass1(mem_t, Lm, Lsq, par_w, pw, par_r, pr):
    big = pl.BlockSpec((BB, D, NB), lambda i, j: (i, 0, j))
    lspec = pl.BlockSpec((BB, 8, D), lambda i, j: (i, 0, 0))
    pspec = pl.BlockSpec((BB, 128), lambda i, j: (i, 0))
    rowspec = pl.BlockSpec((BB, N), lambda i, j: (i, 0))
    return pl.pallas_call(
        _pass1_kernel,
        grid=(GB, GJ),
        in_specs=[big, lspec, lspec, pspec, rowspec, pspec, rowspec],
        out_specs=[rowspec, rowspec,
                   pl.BlockSpec((BB, 1), lambda i, j: (i, 0))],
        out_shape=[jax.ShapeDtypeStruct((B, N), jnp.float32),
                   jax.ShapeDtypeStruct((B, N), jnp.float32),
                   jax.ShapeDtypeStruct((B, 1), jnp.float32)],
        scratch_shapes=[pltpu.VMEM((8, BB, N), jnp.float32)],
        compiler_params=pltpu.CompilerParams(
            dimension_semantics=("parallel", "arbitrary")),
    )(mem_t, Lm, Lsq, par_w, pw, par_r, pr)


def _address(dots, sqn, par, pw):
    beta = par[:, 0:1]
    g = par[:, 1:2]
    gamma = par[:, 2:3]
    s0 = par[:, 3:4]
    s1 = par[:, 4:5]
    s2 = par[:, 5:6]
    knorm = jnp.sqrt(par[:, 6:7])
    norms = jnp.sqrt(jnp.maximum(sqn, 0.0)) * knorm
    z = beta * (dots / (norms + EPS))
    zmax = jnp.max(z, axis=-1, keepdims=True)
    ez = jnp.exp(z - zmax)
    wc = ez / jnp.sum(ez, axis=-1, keepdims=True)
    wg = g * wc + (1.0 - g) * pw
    roll_m1 = jnp.concatenate([wg[:, 1:], wg[:, :1]], axis=-1)
    roll_p1 = jnp.concatenate([wg[:, -1:], wg[:, :-1]], axis=-1)
    ws = s0 * roll_m1 + s1 * wg + s2 * roll_p1
    u = jnp.exp(gamma * jnp.log(ws))
    return u / (jnp.sum(u, axis=-1, keepdims=True) + EPS)


def _pass2_kernel(mem_ref, wr_ref, wrww_ref, swr_ref, erase_ref, add_ref,
                  out_ref, acc1_ref, acc2_ref):
    j = pl.program_id(1)

    @pl.when(j == 0)
    def _init():
        acc1_ref[...] = jnp.zeros_like(acc1_ref)
        acc2_ref[...] = jnp.zeros_like(acc2_ref)

    off = pl.multiple_of(j * NB, NB)
    for b in range(BB):
        mb = mem_ref[b]
        w1 = wr_ref[b:b + 1, pl.ds(off, NB)]
        w2 = wrww_ref[b:b + 1, pl.ds(off, NB)]
        p1 = mb * w1
        p2 = mb * w2
        f1 = sum([p1[:, k * 128:(k + 1) * 128] for k in range(NB // 128)])
        f2 = sum([p2[:, k * 128:(k + 1) * 128] for k in range(NB // 128)])
        acc1_ref[b] += f1
        acc2_ref[b] += f2

    @pl.when(j == GJ - 1)
    def _fin():
        cols1 = [jnp.sum(acc1_ref[b], axis=-1, keepdims=True)
                 for b in range(BB)]
        cols2 = [jnp.sum(acc2_ref[b], axis=-1, keepdims=True)
                 for b in range(BB)]
        p1t = jnp.concatenate(cols1, axis=-1).T
        p2t = jnp.concatenate(cols2, axis=-1).T
        out_ref[...] = (p1t - erase_ref[...] * p2t
                        + swr_ref[...] * add_ref[...])


def _run_pass2(mem_t, wr, wrww, swr, erase, add):
    big = pl.BlockSpec((BB, D, NB), lambda i, j: (i, 0, j))
    rowspec = pl.BlockSpec((BB, N), lambda i, j: (i, 0))
    vec = pl.BlockSpec((BB, D), lambda i, j: (i, 0))
    return pl.pallas_call(
        _pass2_kernel,
        grid=(GB, GJ),
        in_specs=[big, rowspec, rowspec,
                  pl.BlockSpec((BB, 1), lambda i, j: (i, 0)), vec, vec],
        out_specs=vec,
        out_shape=jax.ShapeDtypeStruct((B, D), jnp.float32),
        scratch_shapes=[pltpu.VMEM((BB, D, 128), jnp.float32),
                        pltpu.VMEM((BB, D, 128), jnp.float32)],
        compiler_params=pltpu.CompilerParams(
            dimension_semantics=("parallel", "arbitrary")),
    )(mem_t, wr, wrww, swr, erase, add)


@jax.jit
def kernel(x, prev_memory, prev_read_weights, prev_write_weights,
           prev_read_vector,
           W_ctrl, b_ctrl,
           Wk_r, bk_r, Wb_r, bb_r, Wg_r, bg_r, Ws_r, bs_r, Wgam_r, bgam_r,
           Wk_w, bk_w, Wb_w, bb_w, Wg_w, bg_w, Ws_w, bs_w, Wgam_w, bgam_w,
           We_w, be_w, Wa_w, ba_w):
    ctrl_in = jnp.concatenate([x, prev_read_vector], axis=-1)

    def pack_scalar_weights(Wb, Wg, Wgam, Ws, bb, bg, bgam, bs):
        Wsc = jnp.concatenate([Wb, Wg, Wgam, Ws], axis=-1)
        Wsc = jnp.pad(Wsc, ((0, 0), (0, 122)))
        bsc = jnp.concatenate([bb, bg, bgam, bs], axis=-1)
        bsc = jnp.pad(bsc, (0, 122)).reshape(1, 128)
        return Wsc, bsc

    Wsc_r, bsc_r = pack_scalar_weights(Wb_r, Wg_r, Wgam_r, Ws_r,
                                       bb_r, bg_r, bgam_r, bs_r)
    Wsc_w, bsc_w = pack_scalar_weights(Wb_w, Wg_w, Wgam_w, Ws_w,
                                       bb_w, bg_w, bgam_w, bs_w)

    h, erase, add, Lm, Lsq, par_r, par_w = _run_prologue(
        ctrl_in, W_ctrl, b_ctrl.reshape(1, C),
        Wk_r, bk_r.reshape(1, D), Wk_w, bk_w.reshape(1, D),
        We_w, be_w.reshape(1, D), Wa_w, ba_w.reshape(1, D),
        Wsc_r, bsc_r, Wsc_w, bsc_w)

    mem_t = jnp.transpose(prev_memory, (0, 2, 1))

    wr, wrww, swr = _run_pass1(mem_t, Lm, Lsq, par_w, prev_write_weights,
                               par_r, prev_read_weights)
    read_vec = _run_pass2(mem_t, wr, wrww, swr, erase, add)
    return jnp.concatenate([h, read_vec], axis=-1)


# speedup vs baseline: 1.1566x; 1.0324x over previous
import jax
import jax.numpy as jnp
from jax import lax
from jax.experimental import pallas as pl
from jax.experimental.pallas import tpu as pltpu

B, N, D, C, IN, S = 64, 8192, 64, 256, 128, 3
CTRL_IN = IN + D
EPS = 1e-8

BB = 8
NB = 4096
GB = B // BB
GJ = N // NB

_DNK = (((1,), (0,)), ((), ()))


def _prologue_kernel(ctrl_in_ref, W_ctrl_ref, b_ctrl_ref,
                     Wk_r_ref, bk_r_ref, Wk_w_ref, bk_w_ref,
                     We_w_ref, be_w_ref, Wa_w_ref, ba_w_ref,
                     Wsc_r_ref, bsc_r_ref, Wsc_w_ref, bsc_w_ref,
                     h_ref, erase_ref, add_ref, Lm_ref, Lsq_ref,
                     par_r_ref, par_w_ref):
    f32 = jnp.float32
    h = jnp.maximum(
        jnp.dot(ctrl_in_ref[...], W_ctrl_ref[...],
                preferred_element_type=f32) + b_ctrl_ref[...], 0.0)
    h_ref[...] = h
    k_r = jnp.dot(h, Wk_r_ref[...], preferred_element_type=f32) + bk_r_ref[...]
    k_w = jnp.dot(h, Wk_w_ref[...], preferred_element_type=f32) + bk_w_ref[...]
    e = jax.nn.sigmoid(
        jnp.dot(h, We_w_ref[...], preferred_element_type=f32) + be_w_ref[...])
    a = jnp.tanh(
        jnp.dot(h, Wa_w_ref[...], preferred_element_type=f32) + ba_w_ref[...])
    erase_ref[...] = e
    add_ref[...] = a

    def pack_rows(rows):
        rs = [v[:, None, :] for v in rows]
        pad = jnp.zeros((B, 8 - len(rows), D), f32)
        return jnp.concatenate(rs + [pad], axis=1).astype(jnp.bfloat16)

    Lm_ref[...] = pack_rows([k_w, k_r, e * k_r, a, a * e])
    Lsq_ref[...] = pack_rows([jnp.ones((B, D), f32), e, e * e])

    ksq_r = jnp.sum(k_r * k_r, axis=-1, keepdims=True)
    ksq_w = jnp.sum(k_w * k_w, axis=-1, keepdims=True)
    ak = jnp.sum(a * k_r, axis=-1, keepdims=True)
    asq = jnp.sum(a * a, axis=-1, keepdims=True)
    for Wsc_ref, bsc_ref, ksq, extra, par_ref in (
            (Wsc_r_ref, bsc_r_ref, ksq_r, [ak, asq], par_r_ref),
            (Wsc_w_ref, bsc_w_ref, ksq_w, [], par_w_ref)):
        raw = (jnp.dot(h, Wsc_ref[...], preferred_element_type=f32)
               + bsc_ref[...])
        beta = jax.nn.softplus(raw[:, 0:1])
        g = jax.nn.sigmoid(raw[:, 1:2])
        gamma = jax.nn.softplus(raw[:, 2:3]) + 1.0
        slog = raw[:, 3:6]
        smax = jnp.max(slog, axis=-1, keepdims=True)
        sexp = jnp.exp(slog - smax)
        s = sexp / jnp.sum(sexp, axis=-1, keepdims=True)
        cols = [beta, g, gamma, s, ksq] + extra
        used = 7 + len(extra)
        cols.append(jnp.zeros((B, 128 - used), f32))
        par_ref[...] = jnp.concatenate(cols, axis=-1)


def _run_prologue(ctrl_in, W_ctrl, b_ctrl, Wk_r, bk_r, Wk_w, bk_w,
                  We_w, be_w, Wa_w, ba_w, Wsc_r, bsc_r, Wsc_w, bsc_w):
    out_shapes = (
        jax.ShapeDtypeStruct((B, C), jnp.float32),
        jax.ShapeDtypeStruct((B, D), jnp.float32),
        jax.ShapeDtypeStruct((B, D), jnp.float32),
        jax.ShapeDtypeStruct((B, 8, D), jnp.bfloat16),
        jax.ShapeDtypeStruct((B, 8, D), jnp.bfloat16),
        jax.ShapeDtypeStruct((B, 128), jnp.float32),
        jax.ShapeDtypeStruct((B, 128), jnp.float32),
    )
    return pl.pallas_call(
        _prologue_kernel,
        out_shape=out_shapes,
    )(ctrl_in, W_ctrl, b_ctrl, Wk_r, bk_r, Wk_w, bk_w,
      We_w, be_w, Wa_w, ba_w, Wsc_r, bsc_r, Wsc_w, bsc_w)


def _mega_kernel(mem_ref, Lm_ref, Lsq_ref, par_w_ref, pw_ref,
                 par_r_ref, pr_ref, erase_ref, add_ref,
                 out_ref, fam_s, wr_s, wrww_s, swr_s, acc1_ref, acc2_ref):
    f32 = jnp.float32
    j = pl.program_id(1)
    jj = lax.rem(j, GJ)
    off = pl.multiple_of(jj * NB, NB)

    @pl.when(j < GJ)
    def _family_phase():
        for b in range(BB):
            mb = mem_ref[b].astype(jnp.bfloat16)
            sq = mb * mb
            om = lax.dot_general(Lm_ref[b], mb, _DNK,
                                 preferred_element_type=f32)
            osq = lax.dot_general(Lsq_ref[b], sq, _DNK,
                                  preferred_element_type=f32)
            fam_s[0, b:b + 1, pl.ds(off, NB)] = om[0:1, :]
            fam_s[1, b:b + 1, pl.ds(off, NB)] = om[1:2, :]
            fam_s[2, b:b + 1, pl.ds(off, NB)] = om[2:3, :]
            fam_s[3, b:b + 1, pl.ds(off, NB)] = om[3:4, :]
            fam_s[4, b:b + 1, pl.ds(off, NB)] = om[4:5, :]
            fam_s[5, b:b + 1, pl.ds(off, NB)] = osq[0:1, :]
            fam_s[6, b:b + 1, pl.ds(off, NB)] = osq[1:2, :]
            fam_s[7, b:b + 1, pl.ds(off, NB)] = osq[2:3, :]

    @pl.when(j == GJ - 1)
    def _finalize_heads():
        par_r = par_r_ref[...]
        ak = par_r[:, 7:8]
        asq = par_r[:, 8:9]
        ww = _address(fam_s[0], fam_s[5], par_w_ref[...], pw_ref[...])
        wwsq = ww * ww
        dots = fam_s[1] - ww * fam_s[2] + ww * ak
        sqn = (fam_s[5] - 2.0 * ww * fam_s[6] + wwsq * fam_s[7]
               + 2.0 * ww * fam_s[3] - 2.0 * wwsq * fam_s[4] + wwsq * asq)
        wr = _address(dots, sqn, par_r, pr_ref[...])
        wr_s[...] = wr
        p = wr * ww
        wrww_s[...] = p
        swr_s[...] = jnp.sum(p, axis=-1, keepdims=True)
        acc1_ref[...] = jnp.zeros_like(acc1_ref)
        acc2_ref[...] = jnp.zeros_like(acc2_ref)

    @pl.when(j >= GJ)
    def _read_phase():
        for b in range(BB):
            mb = mem_ref[b]
            w1 = wr_s[b:b + 1, pl.ds(off, NB)]
            w2 = wrww_s[b:b + 1, pl.ds(off, NB)]
            p1 = mb * w1
            p2 = mb * w2
            f1 = sum([p1[:, k * 128:(k + 1) * 128] for k in range(NB // 128)])
            f2 = sum([p2[:, k * 128:(k + 1) * 128] for k in range(NB // 128)])
            acc1_ref[b] += f1
            acc2_ref[b] += f2

    @pl.when(j == 2 * GJ - 1)
    def _fin():
        cols1 = [jnp.sum(acc1_ref[b], axis=-1, keepdims=True)
                 for b in range(BB)]
        cols2 = [jnp.sum(acc2_ref[b], axis=-1, keepdims=True)
                 for b in range(BB)]
        p1t = jnp.concatenate(cols1, axis=-1).T
        p2t = jnp.concatenate(cols2, axis=-1).T
        out_ref[...] = (p1t - erase_ref[...] * p2t
                        + swr_s[...] * add_ref[...])


def _run_mega(mem_t, Lm, Lsq, par_w, pw, par_r, pr, erase, add):
    big = pl.BlockSpec((BB, D, NB), lambda i, j: (i, 0, j % GJ))
    lspec = pl.BlockSpec((BB, 8, D), lambda i, j: (i, 0, 0))
    pspec = pl.BlockSpec((BB, 128), lambda i, j: (i, 0))
    rowspec = pl.BlockSpec((BB, N), lambda i, j: (i, 0))
    vec = pl.BlockSpec((BB, D), lambda i, j: (i, 0))
    return pl.pallas_call(
        _mega_kernel,
        grid=(GB, 2 * GJ),
        in_specs=[big, lspec, lspec, pspec, rowspec, pspec, rowspec,
                  vec, vec],
        out_specs=vec,
        out_shape=jax.ShapeDtypeStruct((B, D), jnp.float32),
        scratch_shapes=[pltpu.VMEM((8, BB, N), jnp.float32),
                        pltpu.VMEM((BB, N), jnp.float32),
                        pltpu.VMEM((BB, N), jnp.float32),
                        pltpu.VMEM((BB, 1), jnp.float32),
                        pltpu.VMEM((BB, D, 128), jnp.float32),
                        pltpu.VMEM((BB, D, 128), jnp.float32)],
        compiler_params=pltpu.CompilerParams(
            dimension_semantics=("parallel", "arbitrary")),
    )(mem_t, Lm, Lsq, par_w, pw, par_r, pr, erase, add)


def _address(dots, sqn, par, pw):
    beta = par[:, 0:1]
    g = par[:, 1:2]
    gamma = par[:, 2:3]
    s0 = par[:, 3:4]
    s1 = par[:, 4:5]
    s2 = par[:, 5:6]
    knorm = jnp.sqrt(par[:, 6:7])
    norms = jnp.sqrt(jnp.maximum(sqn, 0.0)) * knorm
    z = beta * (dots / (norms + EPS))
    zmax = jnp.max(z, axis=-1, keepdims=True)
    ez = jnp.exp(z - zmax)
    wc = ez / jnp.sum(ez, axis=-1, keepdims=True)
    wg = g * wc + (1.0 - g) * pw
    roll_m1 = jnp.concatenate([wg[:, 1:], wg[:, :1]], axis=-1)
    roll_p1 = jnp.concatenate([wg[:, -1:], wg[:, :-1]], axis=-1)
    ws = s0 * roll_m1 + s1 * wg + s2 * roll_p1
    u = jnp.exp(gamma * jnp.log(ws))
    return u / (jnp.sum(u, axis=-1, keepdims=True) + EPS)


def _pass2_kernel(mem_ref, wr_ref, wrww_ref, swr_ref, erase_ref, add_ref,
                  out_ref, acc1_ref, acc2_ref):
    j = pl.program_id(1)

    @pl.when(j == 0)
    def _init():
        acc1_ref[...] = jnp.zeros_like(acc1_ref)
        acc2_ref[...] = jnp.zeros_like(acc2_ref)

    off = pl.multiple_of(j * NB, NB)
    for b in range(BB):
        mb = mem_ref[b]
        w1 = wr_ref[b:b + 1, pl.ds(off, NB)]
        w2 = wrww_ref[b:b + 1, pl.ds(off, NB)]
        p1 = mb * w1
        p2 = mb * w2
        f1 = sum([p1[:, k * 128:(k + 1) * 128] for k in range(NB // 128)])
        f2 = sum([p2[:, k * 128:(k + 1) * 128] for k in range(NB // 128)])
        acc1_ref[b] += f1
        acc2_ref[b] += f2

    @pl.when(j == GJ - 1)
    def _fin():
        cols1 = [jnp.sum(acc1_ref[b], axis=-1, keepdims=True)
                 for b in range(BB)]
        cols2 = [jnp.sum(acc2_ref[b], axis=-1, keepdims=True)
                 for b in range(BB)]
        p1t = jnp.concatenate(cols1, axis=-1).T
        p2t = jnp.concatenate(cols2, axis=-1).T
        out_ref[...] = (p1t - erase_ref[...] * p2t
                        + swr_ref[...] * add_ref[...])


def _run_pass2(mem_t, wr, wrww, swr, erase, add):
    big = pl.BlockSpec((BB, D, NB), lambda i, j: (i, 0, j))
    rowspec = pl.BlockSpec((BB, N), lambda i, j: (i, 0))
    vec = pl.BlockSpec((BB, D), lambda i, j: (i, 0))
    return pl.pallas_call(
        _pass2_kernel,
        grid=(GB, GJ),
        in_specs=[big, rowspec, rowspec,
                  pl.BlockSpec((BB, 1), lambda i, j: (i, 0)), vec, vec],
        out_specs=vec,
        out_shape=jax.ShapeDtypeStruct((B, D), jnp.float32),
        scratch_shapes=[pltpu.VMEM((BB, D, 128), jnp.float32),
                        pltpu.VMEM((BB, D, 128), jnp.float32)],
        compiler_params=pltpu.CompilerParams(
            dimension_semantics=("parallel", "arbitrary")),
    )(mem_t, wr, wrww, swr, erase, add)


@jax.jit
def kernel(x, prev_memory, prev_read_weights, prev_write_weights,
           prev_read_vector,
           W_ctrl, b_ctrl,
           Wk_r, bk_r, Wb_r, bb_r, Wg_r, bg_r, Ws_r, bs_r, Wgam_r, bgam_r,
           Wk_w, bk_w, Wb_w, bb_w, Wg_w, bg_w, Ws_w, bs_w, Wgam_w, bgam_w,
           We_w, be_w, Wa_w, ba_w):
    ctrl_in = jnp.concatenate([x, prev_read_vector], axis=-1)

    def pack_scalar_weights(Wb, Wg, Wgam, Ws, bb, bg, bgam, bs):
        Wsc = jnp.concatenate([Wb, Wg, Wgam, Ws], axis=-1)
        Wsc = jnp.pad(Wsc, ((0, 0), (0, 122)))
        bsc = jnp.concatenate([bb, bg, bgam, bs], axis=-1)
        bsc = jnp.pad(bsc, (0, 122)).reshape(1, 128)
        return Wsc, bsc

    Wsc_r, bsc_r = pack_scalar_weights(Wb_r, Wg_r, Wgam_r, Ws_r,
                                       bb_r, bg_r, bgam_r, bs_r)
    Wsc_w, bsc_w = pack_scalar_weights(Wb_w, Wg_w, Wgam_w, Ws_w,
                                       bb_w, bg_w, bgam_w, bs_w)

    h, erase, add, Lm, Lsq, par_r, par_w = _run_prologue(
        ctrl_in, W_ctrl, b_ctrl.reshape(1, C),
        Wk_r, bk_r.reshape(1, D), Wk_w, bk_w.reshape(1, D),
        We_w, be_w.reshape(1, D), Wa_w, ba_w.reshape(1, D),
        Wsc_r, bsc_r, Wsc_w, bsc_w)

    mem_t = jnp.transpose(prev_memory, (0, 2, 1))

    read_vec = _run_mega(mem_t, Lm, Lsq, par_w, prev_write_weights,
                         par_r, prev_read_weights, erase, add)
    return jnp.concatenate([h, read_vec], axis=-1)


# speedup vs baseline: 1.1596x; 1.0025x over previous
import jax
import jax.numpy as jnp
from jax import lax
from jax.experimental import pallas as pl
from jax.experimental.pallas import tpu as pltpu

B, N, D, C, IN, S = 64, 8192, 64, 256, 128, 3
CTRL_IN = IN + D
EPS = 1e-8

BB = 8
NB = 4096
GB = B // BB
GJ = N // NB

_DNK = (((1,), (0,)), ((), ()))


def _prologue_kernel(ctrl_in_ref, W_ctrl_ref, b_ctrl_ref,
                     Wk_r_ref, bk_r_ref, Wk_w_ref, bk_w_ref,
                     We_w_ref, be_w_ref, Wa_w_ref, ba_w_ref,
                     Wsc_r_ref, bsc_r_ref, Wsc_w_ref, bsc_w_ref,
                     h_ref, erase_ref, add_ref, Lm_ref, Lsq_ref,
                     par_r_ref, par_w_ref):
    f32 = jnp.float32
    h = jnp.maximum(
        jnp.dot(ctrl_in_ref[...], W_ctrl_ref[...],
                preferred_element_type=f32) + b_ctrl_ref[...], 0.0)
    h_ref[...] = h
    k_r = jnp.dot(h, Wk_r_ref[...], preferred_element_type=f32) + bk_r_ref[...]
    k_w = jnp.dot(h, Wk_w_ref[...], preferred_element_type=f32) + bk_w_ref[...]
    e = jax.nn.sigmoid(
        jnp.dot(h, We_w_ref[...], preferred_element_type=f32) + be_w_ref[...])
    a = jnp.tanh(
        jnp.dot(h, Wa_w_ref[...], preferred_element_type=f32) + ba_w_ref[...])
    erase_ref[...] = e
    add_ref[...] = a

    def pack_rows(rows):
        rs = [v[:, None, :] for v in rows]
        pad = jnp.zeros((B, 8 - len(rows), D), f32)
        return jnp.concatenate(rs + [pad], axis=1).astype(jnp.bfloat16)

    Lm_ref[...] = pack_rows([k_w, k_r, e * k_r, a, a * e])
    Lsq_ref[...] = pack_rows([jnp.ones((B, D), f32), e, e * e])

    ksq_r = jnp.sum(k_r * k_r, axis=-1, keepdims=True)
    ksq_w = jnp.sum(k_w * k_w, axis=-1, keepdims=True)
    ak = jnp.sum(a * k_r, axis=-1, keepdims=True)
    asq = jnp.sum(a * a, axis=-1, keepdims=True)
    for Wsc_ref, bsc_ref, ksq, extra, par_ref in (
            (Wsc_r_ref, bsc_r_ref, ksq_r, [ak, asq], par_r_ref),
            (Wsc_w_ref, bsc_w_ref, ksq_w, [], par_w_ref)):
        raw = (jnp.dot(h, Wsc_ref[...], preferred_element_type=f32)
               + bsc_ref[...])
        beta = jax.nn.softplus(raw[:, 0:1])
        g = jax.nn.sigmoid(raw[:, 1:2])
        gamma = jax.nn.softplus(raw[:, 2:3]) + 1.0
        slog = raw[:, 3:6]
        smax = jnp.max(slog, axis=-1, keepdims=True)
        sexp = jnp.exp(slog - smax)
        s = sexp / jnp.sum(sexp, axis=-1, keepdims=True)
        cols = [beta, g, gamma, s, ksq] + extra
        used = 7 + len(extra)
        cols.append(jnp.zeros((B, 128 - used), f32))
        par_ref[...] = jnp.concatenate(cols, axis=-1)


def _run_prologue(ctrl_in, W_ctrl, b_ctrl, Wk_r, bk_r, Wk_w, bk_w,
                  We_w, be_w, Wa_w, ba_w, Wsc_r, bsc_r, Wsc_w, bsc_w):
    out_shapes = (
        jax.ShapeDtypeStruct((B, C), jnp.float32),
        jax.ShapeDtypeStruct((B, D), jnp.float32),
        jax.ShapeDtypeStruct((B, D), jnp.float32),
        jax.ShapeDtypeStruct((B, 8, D), jnp.bfloat16),
        jax.ShapeDtypeStruct((B, 8, D), jnp.bfloat16),
        jax.ShapeDtypeStruct((B, 128), jnp.float32),
        jax.ShapeDtypeStruct((B, 128), jnp.float32),
    )
    return pl.pallas_call(
        _prologue_kernel,
        out_shape=out_shapes,
    )(ctrl_in, W_ctrl, b_ctrl, Wk_r, bk_r, Wk_w, bk_w,
      We_w, be_w, Wa_w, ba_w, Wsc_r, bsc_r, Wsc_w, bsc_w)


def _mega_kernel(mem_ref, Lm_ref, Lsq_ref, par_w_ref, pw_ref,
                 par_r_ref, pr_ref, erase_ref, add_ref,
                 out_ref, fam_s, wr_s, wrww_s, swr_s, acc1_ref, acc2_ref):
    f32 = jnp.float32
    j = pl.program_id(1)
    jj = lax.rem(j, GJ)
    off = pl.multiple_of(jj * NB, NB)

    @pl.when(j < GJ)
    def _family_phase():
        for b in range(BB):
            mb = mem_ref[b].astype(jnp.bfloat16)
            sq = mb * mb
            om = lax.dot_general(Lm_ref[b], mb, _DNK,
                                 preferred_element_type=f32)
            osq = lax.dot_general(Lsq_ref[b], sq, _DNK,
                                  preferred_element_type=f32)
            fam_s[0, b:b + 1, pl.ds(off, NB)] = om[0:1, :]
            fam_s[1, b:b + 1, pl.ds(off, NB)] = om[1:2, :]
            fam_s[2, b:b + 1, pl.ds(off, NB)] = om[2:3, :]
            fam_s[3, b:b + 1, pl.ds(off, NB)] = om[3:4, :]
            fam_s[4, b:b + 1, pl.ds(off, NB)] = om[4:5, :]
            fam_s[5, b:b + 1, pl.ds(off, NB)] = osq[0:1, :]
            fam_s[6, b:b + 1, pl.ds(off, NB)] = osq[1:2, :]
            fam_s[7, b:b + 1, pl.ds(off, NB)] = osq[2:3, :]

    @pl.when(j == GJ - 1)
    def _finalize_heads():
        par_r = par_r_ref[...]
        ak = par_r[:, 7:8]
        asq = par_r[:, 8:9]
        ww = _address(fam_s[0], fam_s[5], par_w_ref[...], pw_ref[...])
        wwsq = ww * ww
        dots = fam_s[1] - ww * fam_s[2] + ww * ak
        sqn = (fam_s[5] - 2.0 * ww * fam_s[6] + wwsq * fam_s[7]
               + 2.0 * ww * fam_s[3] - 2.0 * wwsq * fam_s[4] + wwsq * asq)
        wr = _address(dots, sqn, par_r, pr_ref[...])
        wr_s[...] = wr
        p = wr * ww
        wrww_s[...] = p
        swr_s[...] = jnp.sum(p, axis=-1, keepdims=True)
        acc1_ref[...] = jnp.zeros_like(acc1_ref)
        acc2_ref[...] = jnp.zeros_like(acc2_ref)

    @pl.when(j >= GJ)
    def _read_phase():
        for b in range(BB):
            mb = mem_ref[b]
            w1 = wr_s[b:b + 1, pl.ds(off, NB)]
            w2 = wrww_s[b:b + 1, pl.ds(off, NB)]
            p1 = mb * w1
            p2 = mb * w2
            f1 = sum([p1[:, k * 128:(k + 1) * 128] for k in range(NB // 128)])
            f2 = sum([p2[:, k * 128:(k + 1) * 128] for k in range(NB // 128)])
            acc1_ref[b] += f1
            acc2_ref[b] += f2

    @pl.when(j == 2 * GJ - 1)
    def _fin():
        cols1 = [jnp.sum(acc1_ref[b], axis=-1, keepdims=True)
                 for b in range(BB)]
        cols2 = [jnp.sum(acc2_ref[b], axis=-1, keepdims=True)
                 for b in range(BB)]
        p1t = jnp.concatenate(cols1, axis=-1).T
        p2t = jnp.concatenate(cols2, axis=-1).T
        out_ref[...] = (p1t - erase_ref[...] * p2t
                        + swr_s[...] * add_ref[...])


def _run_mega(mem_t, Lm, Lsq, par_w, pw, par_r, pr, erase, add):
    big = pl.BlockSpec((BB, D, NB), lambda i, j: (i, 0, j % GJ))
    lspec = pl.BlockSpec((BB, 8, D), lambda i, j: (i, 0, 0))
    pspec = pl.BlockSpec((BB, 128), lambda i, j: (i, 0))
    rowspec = pl.BlockSpec((BB, N), lambda i, j: (i, 0))
    vec = pl.BlockSpec((BB, D), lambda i, j: (i, 0))
    return pl.pallas_call(
        _mega_kernel,
        grid=(GB, 2 * GJ),
        in_specs=[big, lspec, lspec, pspec, rowspec, pspec, rowspec,
                  vec, vec],
        out_specs=vec,
        out_shape=jax.ShapeDtypeStruct((B, D), jnp.float32),
        scratch_shapes=[pltpu.VMEM((8, BB, N), jnp.float32),
                        pltpu.VMEM((BB, N), jnp.float32),
                        pltpu.VMEM((BB, N), jnp.float32),
                        pltpu.VMEM((BB, 1), jnp.float32),
                        pltpu.VMEM((BB, D, 128), jnp.float32),
                        pltpu.VMEM((BB, D, 128), jnp.float32)],
        compiler_params=pltpu.CompilerParams(
            dimension_semantics=("parallel", "arbitrary")),
    )(mem_t, Lm, Lsq, par_w, pw, par_r, pr, erase, add)


def _address(dots, sqn, par, pw):
    beta = par[:, 0:1]
    g = par[:, 1:2]
    gamma = par[:, 2:3]
    s0 = par[:, 3:4]
    s1 = par[:, 4:5]
    s2 = par[:, 5:6]
    knorm = jnp.sqrt(par[:, 6:7])
    norms = jnp.sqrt(jnp.maximum(sqn, 0.0)) * knorm
    z = beta * (dots / (norms + EPS))
    zmax = jnp.max(z, axis=-1, keepdims=True)
    ez = jnp.exp(z - zmax)
    wc = ez / jnp.sum(ez, axis=-1, keepdims=True)
    wg = g * wc + (1.0 - g) * pw
    roll_m1 = jnp.concatenate([wg[:, 1:], wg[:, :1]], axis=-1)
    roll_p1 = jnp.concatenate([wg[:, -1:], wg[:, :-1]], axis=-1)
    ws = s0 * roll_m1 + s1 * wg + s2 * roll_p1
    u = jnp.exp(gamma * jnp.log(ws))
    return u / (jnp.sum(u, axis=-1, keepdims=True) + EPS)


@jax.jit
def kernel(x, prev_memory, prev_read_weights, prev_write_weights,
           prev_read_vector,
           W_ctrl, b_ctrl,
           Wk_r, bk_r, Wb_r, bb_r, Wg_r, bg_r, Ws_r, bs_r, Wgam_r, bgam_r,
           Wk_w, bk_w, Wb_w, bb_w, Wg_w, bg_w, Ws_w, bs_w, Wgam_w, bgam_w,
           We_w, be_w, Wa_w, ba_w):
    ctrl_in = jnp.concatenate([x, prev_read_vector], axis=-1)

    def pack_scalar_weights(Wb, Wg, Wgam, Ws, bb, bg, bgam, bs):
        Wsc = jnp.concatenate([Wb, Wg, Wgam, Ws], axis=-1)
        Wsc = jnp.pad(Wsc, ((0, 0), (0, 122)))
        bsc = jnp.concatenate([bb, bg, bgam, bs], axis=-1)
        bsc = jnp.pad(bsc, (0, 122)).reshape(1, 128)
        return Wsc, bsc

    Wsc_r, bsc_r = pack_scalar_weights(Wb_r, Wg_r, Wgam_r, Ws_r,
                                       bb_r, bg_r, bgam_r, bs_r)
    Wsc_w, bsc_w = pack_scalar_weights(Wb_w, Wg_w, Wgam_w, Ws_w,
                                       bb_w, bg_w, bgam_w, bs_w)

    h, erase, add, Lm, Lsq, par_r, par_w = _run_prologue(
        ctrl_in, W_ctrl, b_ctrl.reshape(1, C),
        Wk_r, bk_r.reshape(1, D), Wk_w, bk_w.reshape(1, D),
        We_w, be_w.reshape(1, D), Wa_w, ba_w.reshape(1, D),
        Wsc_r, bsc_r, Wsc_w, bsc_w)

    mem_t = jnp.transpose(prev_memory, (0, 2, 1))

    read_vec = _run_mega(mem_t, Lm, Lsq, par_w, prev_write_weights,
                         par_r, prev_read_weights, erase, add)
    return jnp.concatenate([h, read_vec], axis=-1)
